# baseline (speedup 1.0000x reference)
import functools

import jax
import jax.numpy as jnp
from jax import lax
from jax.experimental import pallas as pl
from jax.experimental.pallas import tpu as pltpu
from jax.experimental.pallas import tpu_sc as plsc

_B = 4096
_NUM = 13
_NCAT = 26
_VOCAB = 100000
_H = 64

_NC = 2
_NS = 16
_NW = _NC * _NS
_PER_W = _B * _NCAT // _NW
_CHUNK = 128
_NCHUNK = _PER_W // _CHUNK

_BSTRIPE = _B // _NW

_BB = 512


def _mlp_body(x_ref, w1_ref, b1_ref, w2_ref, b2_ref, out_ref):
    x = x_ref[...]
    for n in range(_NUM):
        h = x[:, n:n + 1] * w1_ref[n:n + 1, :] + b1_ref[n:n + 1, :]
        h = 0.5 * h * (1.0 + lax.erf(h * 0.7071067811865476))
        t = jnp.dot(h, w2_ref[n], preferred_element_type=jnp.float32)
        out_ref[:, n, :] = t + b2_ref[n:n + 1, :]


def _num_tokens(x_num, W1, b1, W2, b2):
    return pl.pallas_call(
        _mlp_body,
        grid=(_B // _BB,),
        in_specs=[
            pl.BlockSpec((_BB, _NUM), lambda i: (i, 0)),
            pl.BlockSpec((_NUM, _H), lambda i: (0, 0)),
            pl.BlockSpec((_NUM, _H), lambda i: (0, 0)),
            pl.BlockSpec((_NUM, _H, _H), lambda i: (0, 0, 0)),
            pl.BlockSpec((_NUM, _H), lambda i: (0, 0)),
        ],
        out_specs=pl.BlockSpec((_BB, _NUM, _H), lambda i: (i, 0, 0)),
        out_shape=jax.ShapeDtypeStruct((_B, _NUM, _H), jnp.float32),
    )(x_num, W1, b1, W2, b2)


def _sc_gather_body(table3d_hbm, idxt_hbm, out_hbm, idx_v, rows_v, sem):
    wid = lax.axis_index("s") * _NC + lax.axis_index("c")
    bstart = pl.multiple_of(wid * _BSTRIPE, _BSTRIPE)
    pltpu.sync_copy(idxt_hbm.at[:, pl.ds(bstart, _BSTRIPE)], idx_v)

    def body(c, carry):
        pltpu.async_copy(table3d_hbm.at[c].at[idx_v.at[c]], rows_v, sem).wait()
        pltpu.sync_copy(rows_v, out_hbm.at[pl.ds(bstart, _BSTRIPE), c])
        return carry

    lax.fori_loop(0, _NCAT, body, 0)


def _sc_gather(tables3d, idx_t):
    mesh = plsc.VectorSubcoreMesh(core_axis_name="c", subcore_axis_name="s")
    run = functools.partial(
        pl.kernel,
        out_type=jax.ShapeDtypeStruct((_B, _NCAT, _H), jnp.float32),
        mesh=mesh,
        scratch_types=[
            pltpu.VMEM((_NCAT, _BSTRIPE), jnp.int32),
            pltpu.VMEM((_BSTRIPE, _H), jnp.float32),
            pltpu.SemaphoreType.DMA,
        ],
        compiler_params=pltpu.CompilerParams(use_tc_tiling_on_sc=False),
    )(_sc_gather_body)
    return run(tables3d, idx_t)


def kernel(x_num, x_cat, W1, b1, W2, b2, tables, cls_token):
    batch = x_num.shape[0]
    cat_tokens = _sc_gather(tables, x_cat.T)
    num_tokens = _num_tokens(x_num, W1, b1, W2, b2)
    cls = jnp.broadcast_to(cls_token, (batch, 1, _H))
    return jnp.concatenate([cls, num_tokens, cat_tokens], axis=1)


# speedup vs baseline: 1.2571x; 1.2571x over previous
import functools

import jax
import jax.numpy as jnp
from jax import lax
from jax.experimental import pallas as pl
from jax.experimental.pallas import tpu as pltpu
from jax.experimental.pallas import tpu_sc as plsc

_B = 4096
_NUM = 13
_NCAT = 26
_VOCAB = 100000
_H = 64

_NC = 2
_NS = 16
_NW = _NC * _NS
_PER_W = _B * _NCAT // _NW
_CHUNK = 128
_NCHUNK = _PER_W // _CHUNK

_VB = 4096
_PPB = _VB // 2
_NVB = (_VOCAB + _VB - 1) // _VB
_PPF = _NVB * _PPB
_BB = 512


def _repack_body(xt_ref, out_ref):
    x = xt_ref[0]
    iden = (lax.broadcasted_iota(jnp.int32, (_H, _H), 0)
            == lax.broadcasted_iota(jnp.int32, (_H, _H), 1)).astype(jnp.float32)
    ya = lax.dot_general(x[:, : _PPB], iden, (((0,), (0,)), ((), ())),
                         preferred_element_type=jnp.float32)
    yb = lax.dot_general(x[:, _PPB:], iden, (((0,), (0,)), ((), ())),
                         preferred_element_type=jnp.float32)
    out_ref[0, :, : _H] = ya
    out_ref[0, :, _H:] = yb


def _repack(tables_t):
    return pl.pallas_call(
        _repack_body,
        grid=(_NCAT, _NVB),
        in_specs=[pl.BlockSpec((1, _H, _VB), lambda c, j: (c, 0, j))],
        out_specs=pl.BlockSpec((1, _PPB, 2 * _H), lambda c, j: (c, j, 0)),
        out_shape=jax.ShapeDtypeStruct((_NCAT, _PPF, 2 * _H), jnp.float32),
    )(tables_t)


def _mlp_body(x_ref, w1_ref, b1_ref, w2_ref, b2_ref, out_ref):
    x = x_ref[...]
    for n in range(_NUM):
        h = x[:, n:n + 1] * w1_ref[n:n + 1, :] + b1_ref[n:n + 1, :]
        h = 0.5 * h * (1.0 + lax.erf(h * 0.7071067811865476))
        t = jnp.dot(h, w2_ref[n], preferred_element_type=jnp.float32)
        out_ref[:, n, :] = t + b2_ref[n:n + 1, :]


def _num_tokens(x_num, W1, b1, W2, b2):
    return pl.pallas_call(
        _mlp_body,
        grid=(_B // _BB,),
        in_specs=[
            pl.BlockSpec((_BB, _NUM), lambda i: (i, 0)),
            pl.BlockSpec((_NUM, _H), lambda i: (0, 0)),
            pl.BlockSpec((_NUM, _H), lambda i: (0, 0)),
            pl.BlockSpec((_NUM, _H, _H), lambda i: (0, 0, 0)),
            pl.BlockSpec((_NUM, _H), lambda i: (0, 0)),
        ],
        out_specs=pl.BlockSpec((_BB, _NUM, _H), lambda i: (i, 0, 0)),
        out_shape=jax.ShapeDtypeStruct((_B, _NUM, _H), jnp.float32),
    )(x_num, W1, b1, W2, b2)


def _sc_gather_body(tpair_hbm, ge_hbm, out_hbm, ge_v, pidx_v, rows_v, out_v,
                    sem):
    wid = lax.axis_index("s") * _NC + lax.axis_index("c")
    base = pl.multiple_of(wid * _PER_W, _PER_W)
    pltpu.sync_copy(ge_hbm.at[pl.ds(base, _PER_W)], ge_v)

    def shift_body(k, carry):
        off = pl.multiple_of(k * 16, 16)
        pidx_v[pl.ds(off, 16)] = lax.shift_right_logical(
            ge_v[pl.ds(off, 16)], 1)
        return carry

    lax.fori_loop(0, _PER_W // 16, shift_body, 0)

    lanes = lax.iota(jnp.int32, 16)

    def chunk_body(j, carry):
        off = pl.multiple_of(j * _CHUNK, _CHUNK)
        pltpu.async_copy(tpair_hbm.at[pidx_v.at[pl.ds(off, _CHUNK)]],
                         rows_v, sem).wait()

        def grp_body(g, carry2):
            goff = pl.multiple_of(g * 16, 16)
            rowi = lanes + goff
            half = lax.bitwise_and(ge_v[pl.ds(off + goff, 16)], 1)
            colbase = half * _H
            for c in range(_H):
                vals = plsc.load_gather(rows_v, [rowi, colbase + c])
                plsc.store_scatter(out_v, [rowi, lanes * 0 + c], vals)
            return carry2

        lax.fori_loop(0, _CHUNK // 16, grp_body, 0)
        pltpu.sync_copy(out_v, out_hbm.at[pl.ds(base + off, _CHUNK)])
        return carry

    lax.fori_loop(0, _NCHUNK, chunk_body, 0)


def _sc_gather(tpair_flat, ge):
    mesh = plsc.VectorSubcoreMesh(core_axis_name="c", subcore_axis_name="s")
    run = functools.partial(
        pl.kernel,
        out_type=jax.ShapeDtypeStruct((_B * _NCAT, _H), jnp.float32),
        mesh=mesh,
        scratch_types=[
            pltpu.VMEM((_PER_W,), jnp.int32),
            pltpu.VMEM((_PER_W,), jnp.int32),
            pltpu.VMEM((_CHUNK, 2 * _H), jnp.float32),
            pltpu.VMEM((_CHUNK, _H), jnp.float32),
            pltpu.SemaphoreType.DMA,
        ],
        compiler_params=pltpu.CompilerParams(use_tc_tiling_on_sc=True,
                                             needs_layout_passes=False),
    )(_sc_gather_body)
    return run(tpair_flat, ge)


def kernel(x_num, x_cat, W1, b1, W2, b2, tables, cls_token):
    batch = x_num.shape[0]
    tables_t = jnp.swapaxes(tables, 1, 2)
    tpair = _repack(tables_t)
    tpair_flat = tpair.reshape(_NCAT * _PPF, 2 * _H)

    v = x_cat
    pair = (v // _VB) * _PPB + (v % _PPB)
    half = (v // _PPB) % 2
    crange = jnp.arange(_NCAT, dtype=jnp.int32)[None, :]
    ge = ((crange * _PPF + pair) * 2 + half).reshape(-1)

    cat_tokens = _sc_gather(tpair_flat, ge).reshape(batch, _NCAT, _H)
    num_tokens = _num_tokens(x_num, W1, b1, W2, b2)
    cls = jnp.broadcast_to(cls_token, (batch, 1, _H))
    return jnp.concatenate([cls, num_tokens, cat_tokens], axis=1)


# speedup vs baseline: 1.2978x; 1.0324x over previous
import functools

import jax
import jax.numpy as jnp
from jax import lax
from jax.experimental import pallas as pl
from jax.experimental.pallas import tpu as pltpu
from jax.experimental.pallas import tpu_sc as plsc

_B = 4096
_NUM = 13
_NCAT = 26
_VOCAB = 100000
_H = 64

_NC = 2
_NS = 16
_NW = _NC * _NS
_PER_W = _B * _NCAT // _NW
_CHUNK = 128
_NCHUNK = _PER_W // _CHUNK

_VB = 4096
_PPB = _VB // 2
_NVB = (_VOCAB + _VB - 1) // _VB
_PPF = _NVB * _PPB
_BB = 512


def _repack_body(xt_ref, out_ref):
    x = xt_ref[0]
    y = jnp.swapaxes(x, 0, 1)
    out_ref[0] = jnp.concatenate([y[: _PPB], y[_PPB:]], axis=1)


def _repack(tables_t):
    return pl.pallas_call(
        _repack_body,
        grid=(_NCAT, _NVB),
        in_specs=[pl.BlockSpec((1, _H, _VB), lambda c, j: (c, 0, j))],
        out_specs=pl.BlockSpec((1, _PPB, 2 * _H), lambda c, j: (c, j, 0)),
        out_shape=jax.ShapeDtypeStruct((_NCAT, _PPF, 2 * _H), jnp.float32),
    )(tables_t)


def _mlp_body(x_ref, w1_ref, b1_ref, w2_ref, b2_ref, out_ref):
    x = x_ref[...]
    for n in range(_NUM):
        h = x[:, n:n + 1] * w1_ref[n:n + 1, :] + b1_ref[n:n + 1, :]
        h = 0.5 * h * (1.0 + lax.erf(h * 0.7071067811865476))
        t = jnp.dot(h, w2_ref[n], preferred_element_type=jnp.float32)
        out_ref[:, n, :] = t + b2_ref[n:n + 1, :]


def _num_tokens(x_num, W1, b1, W2, b2):
    return pl.pallas_call(
        _mlp_body,
        grid=(_B // _BB,),
        in_specs=[
            pl.BlockSpec((_BB, _NUM), lambda i: (i, 0)),
            pl.BlockSpec((_NUM, _H), lambda i: (0, 0)),
            pl.BlockSpec((_NUM, _H), lambda i: (0, 0)),
            pl.BlockSpec((_NUM, _H, _H), lambda i: (0, 0, 0)),
            pl.BlockSpec((_NUM, _H), lambda i: (0, 0)),
        ],
        out_specs=pl.BlockSpec((_BB, _NUM, _H), lambda i: (i, 0, 0)),
        out_shape=jax.ShapeDtypeStruct((_B, _NUM, _H), jnp.float32),
    )(x_num, W1, b1, W2, b2)


def _sc_gather_body(tpair_hbm, ge_hbm, out_hbm, ge_v, pidx_v, rows_v, out_v,
                    sems):
    wid = lax.axis_index("s") * _NC + lax.axis_index("c")
    base = pl.multiple_of(wid * _PER_W, _PER_W)
    pltpu.sync_copy(ge_hbm.at[pl.ds(base, _PER_W)], ge_v)

    def shift_body(k, carry):
        off = pl.multiple_of(k * 16, 16)
        pidx_v[pl.ds(off, 16)] = lax.shift_right_logical(
            ge_v[pl.ds(off, 16)], 1)
        return carry

    lax.fori_loop(0, _PER_W // 16, shift_body, 0)

    lanes = lax.iota(jnp.int32, 16)

    def fire(j, slot):
        off = pl.multiple_of(j * _CHUNK, _CHUNK)
        pltpu.async_copy(tpair_hbm.at[pidx_v.at[pl.ds(off, _CHUNK)]],
                         rows_v.at[slot], sems.at[slot])

    fire(0, 0)

    def chunk_body(j, carry):
        slot = lax.rem(j, 2)
        nxt = lax.rem(j + 1, 2)

        @pl.when(j + 1 < _NCHUNK)
        def _():
            fire(j + 1, nxt)

        pltpu.make_async_copy(tpair_hbm.at[pl.ds(0, _CHUNK)],
                              rows_v.at[slot], sems.at[slot]).wait()

        off = pl.multiple_of(j * _CHUNK, _CHUNK)
        rv = rows_v.at[slot]

        def grp_body(g, carry2):
            goff = pl.multiple_of(g * 16, 16)
            rowi = lanes + goff
            half = lax.bitwise_and(ge_v[pl.ds(off + goff, 16)], 1)
            colbase = half * _H
            for c in range(_H):
                vals = plsc.load_gather(rv, [rowi, colbase + c])
                plsc.store_scatter(out_v, [rowi, lanes * 0 + c], vals)
            return carry2

        lax.fori_loop(0, _CHUNK // 16, grp_body, 0)
        pltpu.sync_copy(out_v, out_hbm.at[pl.ds(base + off, _CHUNK)])
        return carry

    lax.fori_loop(0, _NCHUNK, chunk_body, 0)


def _sc_gather(tpair_flat, ge):
    mesh = plsc.VectorSubcoreMesh(core_axis_name="c", subcore_axis_name="s")
    run = functools.partial(
        pl.kernel,
        out_type=jax.ShapeDtypeStruct((_B * _NCAT, _H), jnp.float32),
        mesh=mesh,
        scratch_types=[
            pltpu.VMEM((_PER_W,), jnp.int32),
            pltpu.VMEM((_PER_W,), jnp.int32),
            pltpu.VMEM((2, _CHUNK, 2 * _H), jnp.float32),
            pltpu.VMEM((_CHUNK, _H), jnp.float32),
            pltpu.SemaphoreType.DMA((2,)),
        ],
        compiler_params=pltpu.CompilerParams(use_tc_tiling_on_sc=True,
                                             needs_layout_passes=False),
    )(_sc_gather_body)
    return run(tpair_flat, ge)


def kernel(x_num, x_cat, W1, b1, W2, b2, tables, cls_token):
    batch = x_num.shape[0]
    tables_t = jnp.swapaxes(tables, 1, 2)
    tpair = _repack(tables_t)
    tpair_flat = tpair.reshape(_NCAT * _PPF, 2 * _H)

    v = x_cat
    pair = (v // _VB) * _PPB + (v % _PPB)
    half = (v // _PPB) % 2
    crange = jnp.arange(_NCAT, dtype=jnp.int32)[None, :]
    ge = ((crange * _PPF + pair) * 2 + half).reshape(-1)

    cat_tokens = _sc_gather(tpair_flat, ge).reshape(batch, _NCAT, _H)
    num_tokens = _num_tokens(x_num, W1, b1, W2, b2)
    cls = jnp.broadcast_to(cls_token, (batch, 1, _H))
    return jnp.concatenate([cls, num_tokens, cat_tokens], axis=1)


# speedup vs baseline: 1.4563x; 1.1221x over previous
import functools

import jax
import jax.numpy as jnp
from jax import lax
from jax.experimental import pallas as pl
from jax.experimental.pallas import tpu as pltpu
from jax.experimental.pallas import tpu_sc as plsc

_B = 4096
_NUM = 13
_NCAT = 26
_VOCAB = 100000
_H = 64

_NC = 2
_NS = 16
_NW = _NC * _NS
_NCH = _NCAT // 2
_PER_W = _B * _NCH // _NW
_CHUNK = 128
_NCHUNK = _PER_W // _CHUNK

_VB = 4096
_PPB = _VB // 2
_NVB = (_VOCAB + _VB - 1) // _VB
_PPF = _NVB * _PPB
_BB = 512


def _repack_body(xt_ref, out_ref):
    x = xt_ref[0]
    y = jnp.swapaxes(x, 0, 1)
    out_ref[0] = jnp.concatenate([y[: _PPB], y[_PPB:]], axis=1)


def _repack(tables_t, c0, nc):
    return pl.pallas_call(
        _repack_body,
        grid=(nc, _NVB),
        in_specs=[pl.BlockSpec((1, _H, _VB), lambda c, j: (c + c0, 0, j))],
        out_specs=pl.BlockSpec((1, _PPB, 2 * _H), lambda c, j: (c, j, 0)),
        out_shape=jax.ShapeDtypeStruct((nc, _PPF, 2 * _H), jnp.float32),
    )(tables_t)


def _mlp_body(x_ref, w1_ref, b1_ref, w2_ref, b2_ref, out_ref):
    x = x_ref[...]
    for n in range(_NUM):
        h = x[:, n:n + 1] * w1_ref[n:n + 1, :] + b1_ref[n:n + 1, :]
        h = 0.5 * h * (1.0 + lax.erf(h * 0.7071067811865476))
        t = jnp.dot(h, w2_ref[n], preferred_element_type=jnp.float32)
        out_ref[:, n, :] = t + b2_ref[n:n + 1, :]


def _num_tokens(x_num, W1, b1, W2, b2):
    return pl.pallas_call(
        _mlp_body,
        grid=(_B // _BB,),
        in_specs=[
            pl.BlockSpec((_BB, _NUM), lambda i: (i, 0)),
            pl.BlockSpec((_NUM, _H), lambda i: (0, 0)),
            pl.BlockSpec((_NUM, _H), lambda i: (0, 0)),
            pl.BlockSpec((_NUM, _H, _H), lambda i: (0, 0, 0)),
            pl.BlockSpec((_NUM, _H), lambda i: (0, 0)),
        ],
        out_specs=pl.BlockSpec((_BB, _NUM, _H), lambda i: (i, 0, 0)),
        out_shape=jax.ShapeDtypeStruct((_B, _NUM, _H), jnp.float32),
    )(x_num, W1, b1, W2, b2)


def _sc_gather_body(tpair_hbm, ge_hbm, out_hbm, ge_v, pidx_v, rows_v, out_v,
                    sems):
    wid = lax.axis_index("s") * _NC + lax.axis_index("c")
    base = pl.multiple_of(wid * _PER_W, _PER_W)
    pltpu.sync_copy(ge_hbm.at[pl.ds(base, _PER_W)], ge_v)

    def shift_body(k, carry):
        off = pl.multiple_of(k * 16, 16)
        pidx_v[pl.ds(off, 16)] = lax.shift_right_logical(
            ge_v[pl.ds(off, 16)], 1)
        return carry

    lax.fori_loop(0, _PER_W // 16, shift_body, 0)

    lanes = lax.iota(jnp.int32, 16)

    def fire(j, slot):
        off = pl.multiple_of(j * _CHUNK, _CHUNK)
        pltpu.async_copy(tpair_hbm.at[pidx_v.at[pl.ds(off, _CHUNK)]],
                         rows_v.at[slot], sems.at[slot])

    fire(0, 0)

    def chunk_body(j, carry):
        slot = lax.rem(j, 2)
        nxt = lax.rem(j + 1, 2)

        @pl.when(j + 1 < _NCHUNK)
        def _():
            fire(j + 1, nxt)

        pltpu.make_async_copy(tpair_hbm.at[pl.ds(0, _CHUNK)],
                              rows_v.at[slot], sems.at[slot]).wait()

        off = pl.multiple_of(j * _CHUNK, _CHUNK)
        rv = rows_v.at[slot]

        def grp_body(g, carry2):
            goff = pl.multiple_of(g * 16, 16)
            rowi = lanes + goff
            half = lax.bitwise_and(ge_v[pl.ds(off + goff, 16)], 1)
            colbase = half * _H
            for c in range(_H):
                vals = plsc.load_gather(rv, [rowi, colbase + c])
                plsc.store_scatter(out_v, [rowi, lanes * 0 + c], vals)
            return carry2

        lax.fori_loop(0, _CHUNK // 16, grp_body, 0)
        pltpu.sync_copy(out_v, out_hbm.at[pl.ds(base + off, _CHUNK)])
        return carry

    lax.fori_loop(0, _NCHUNK, chunk_body, 0)


def _sc_gather(tpair_flat, ge):
    mesh = plsc.VectorSubcoreMesh(core_axis_name="c", subcore_axis_name="s")
    run = functools.partial(
        pl.kernel,
        out_type=jax.ShapeDtypeStruct((_B * _NCH, _H), jnp.float32),
        mesh=mesh,
        scratch_types=[
            pltpu.VMEM((_PER_W,), jnp.int32),
            pltpu.VMEM((_PER_W,), jnp.int32),
            pltpu.VMEM((2, _CHUNK, 2 * _H), jnp.float32),
            pltpu.VMEM((_CHUNK, _H), jnp.float32),
            pltpu.SemaphoreType.DMA((2,)),
        ],
        compiler_params=pltpu.CompilerParams(use_tc_tiling_on_sc=True,
                                             needs_layout_passes=False),
    )(_sc_gather_body)
    return run(tpair_flat, ge)


def kernel(x_num, x_cat, W1, b1, W2, b2, tables, cls_token):
    batch = x_num.shape[0]
    tables_t = jnp.swapaxes(tables, 1, 2)

    v = x_cat
    pair = (v // _VB) * _PPB + (v % _PPB)
    half = (v // _PPB) % 2
    crange = jnp.arange(_NCH, dtype=jnp.int32)[None, :]
    ge1 = ((crange * _PPF + pair[:, : _NCH]) * 2
           + half[:, : _NCH]).reshape(-1)
    ge2 = ((crange * _PPF + pair[:, _NCH:]) * 2
           + half[:, _NCH:]).reshape(-1)

    tp1 = _repack(tables_t, 0, _NCH).reshape(_NCH * _PPF, 2 * _H)
    cat1 = _sc_gather(tp1, ge1).reshape(batch, _NCH, _H)
    tp2 = _repack(tables_t, _NCH, _NCH).reshape(_NCH * _PPF, 2 * _H)
    cat2 = _sc_gather(tp2, ge2).reshape(batch, _NCH, _H)

    num_tokens = _num_tokens(x_num, W1, b1, W2, b2)
    cls = jnp.broadcast_to(cls_token, (batch, 1, _H))
    return jnp.concatenate([cls, num_tokens, cat1, cat2], axis=1)


# speedup vs baseline: 1.6902x; 1.1607x over previous
import functools

import jax
import jax.numpy as jnp
from jax import lax
from jax.experimental import pallas as pl
from jax.experimental.pallas import tpu as pltpu
from jax.experimental.pallas import tpu_sc as plsc

_B = 4096
_NUM = 13
_NCAT = 26
_VOCAB = 100000
_H = 64

_NC = 2
_NS = 16
_NW = _NC * _NS
_NCH = _NCAT // 2
_PER_W = _B * _NCH // _NW
_CHUNK = 128
_NCHUNK = _PER_W // _CHUNK

_VB = 8192
_PPB = _VB // 2
_NVB = (_VOCAB + _VB - 1) // _VB
_PPF = _NVB * _PPB
_BB = 512


def _repack_body(xt_ref, out_ref):
    x = xt_ref[0]
    y = jnp.swapaxes(x, 0, 1)
    out_ref[0] = jnp.concatenate([y[: _PPB], y[_PPB:]], axis=1)


def _repack(tables_t, c0, nc):
    return pl.pallas_call(
        _repack_body,
        grid=(nc, _NVB),
        in_specs=[pl.BlockSpec((1, _H, _VB), lambda c, j: (c + c0, 0, j))],
        out_specs=pl.BlockSpec((1, _PPB, 2 * _H), lambda c, j: (c, j, 0)),
        out_shape=jax.ShapeDtypeStruct((nc, _PPF, 2 * _H), jnp.float32),
    )(tables_t)


def _mlp_body(x_ref, w1_ref, b1_ref, w2_ref, b2_ref, out_ref):
    x = x_ref[...]
    for n in range(_NUM):
        h = x[:, n:n + 1] * w1_ref[n:n + 1, :] + b1_ref[n:n + 1, :]
        h = 0.5 * h * (1.0 + lax.erf(h * 0.7071067811865476))
        t = jnp.dot(h, w2_ref[n], preferred_element_type=jnp.float32)
        out_ref[:, n, :] = t + b2_ref[n:n + 1, :]


def _num_tokens(x_num, W1, b1, W2, b2):
    return pl.pallas_call(
        _mlp_body,
        grid=(_B // _BB,),
        in_specs=[
            pl.BlockSpec((_BB, _NUM), lambda i: (i, 0)),
            pl.BlockSpec((_NUM, _H), lambda i: (0, 0)),
            pl.BlockSpec((_NUM, _H), lambda i: (0, 0)),
            pl.BlockSpec((_NUM, _H, _H), lambda i: (0, 0, 0)),
            pl.BlockSpec((_NUM, _H), lambda i: (0, 0)),
        ],
        out_specs=pl.BlockSpec((_BB, _NUM, _H), lambda i: (i, 0, 0)),
        out_shape=jax.ShapeDtypeStruct((_B, _NUM, _H), jnp.float32),
    )(x_num, W1, b1, W2, b2)


def _sc_gather_body(tpair_hbm, ge_hbm, out_hbm, ge_v, pidx_v, rows_v, out_v,
                    sems):
    wid = lax.axis_index("s") * _NC + lax.axis_index("c")
    base = pl.multiple_of(wid * _PER_W, _PER_W)
    pltpu.sync_copy(ge_hbm.at[pl.ds(base, _PER_W)], ge_v)

    def shift_body(k, carry):
        off = pl.multiple_of(k * 16, 16)
        pidx_v[pl.ds(off, 16)] = lax.shift_right_logical(
            ge_v[pl.ds(off, 16)], 1)
        return carry

    lax.fori_loop(0, _PER_W // 16, shift_body, 0)

    lanes = lax.iota(jnp.int32, 16)

    def fire(j, slot):
        off = pl.multiple_of(j * _CHUNK, _CHUNK)
        pltpu.async_copy(tpair_hbm.at[pidx_v.at[pl.ds(off, _CHUNK)]],
                         rows_v.at[slot], sems.at[slot])

    fire(0, 0)

    def chunk_body(j, carry):
        slot = lax.rem(j, 2)
        nxt = lax.rem(j + 1, 2)

        @pl.when(j + 1 < _NCHUNK)
        def _():
            fire(j + 1, nxt)

        pltpu.make_async_copy(tpair_hbm.at[pl.ds(0, _CHUNK)],
                              rows_v.at[slot], sems.at[slot]).wait()

        off = pl.multiple_of(j * _CHUNK, _CHUNK)
        rv = rows_v.at[slot]

        def grp_body(g, carry2):
            goff = pl.multiple_of(g * 16, 16)
            rowi = lanes + goff
            half = lax.bitwise_and(ge_v[pl.ds(off + goff, 16)], 1)
            colbase = half * _H
            for c in range(_H):
                vals = plsc.load_gather(rv, [rowi, colbase + c])
                plsc.store_scatter(out_v, [rowi, lanes * 0 + c], vals)
            return carry2

        lax.fori_loop(0, _CHUNK // 16, grp_body, 0)
        pltpu.sync_copy(out_v, out_hbm.at[pl.ds(base + off, _CHUNK)])
        return carry

    lax.fori_loop(0, _NCHUNK, chunk_body, 0)


def _sc_gather(tpair_flat, ge):
    mesh = plsc.VectorSubcoreMesh(core_axis_name="c", subcore_axis_name="s")
    run = functools.partial(
        pl.kernel,
        out_type=jax.ShapeDtypeStruct((_B * _NCH, _H), jnp.float32),
        mesh=mesh,
        scratch_types=[
            pltpu.VMEM((_PER_W,), jnp.int32),
            pltpu.VMEM((_PER_W,), jnp.int32),
            pltpu.VMEM((2, _CHUNK, 2 * _H), jnp.float32),
            pltpu.VMEM((_CHUNK, _H), jnp.float32),
            pltpu.SemaphoreType.DMA((2,)),
        ],
        compiler_params=pltpu.CompilerParams(use_tc_tiling_on_sc=True,
                                             needs_layout_passes=False),
    )(_sc_gather_body)
    return run(tpair_flat, ge)


def kernel(x_num, x_cat, W1, b1, W2, b2, tables, cls_token):
    batch = x_num.shape[0]
    tables_t = jnp.swapaxes(tables, 1, 2)

    v = x_cat
    pair = (v // _VB) * _PPB + (v % _PPB)
    half = (v // _PPB) % 2
    crange = jnp.arange(_NCH, dtype=jnp.int32)[None, :]
    ge1 = ((crange * _PPF + pair[:, : _NCH]) * 2
           + half[:, : _NCH]).reshape(-1)
    ge2 = ((crange * _PPF + pair[:, _NCH:]) * 2
           + half[:, _NCH:]).reshape(-1)

    tp1 = _repack(tables_t, 0, _NCH).reshape(_NCH * _PPF, 2 * _H)
    cat1 = _sc_gather(tp1, ge1).reshape(batch, _NCH, _H)
    tp2 = _repack(tables_t, _NCH, _NCH).reshape(_NCH * _PPF, 2 * _H)
    cat2 = _sc_gather(tp2, ge2).reshape(batch, _NCH, _H)

    num_tokens = _num_tokens(x_num, W1, b1, W2, b2)
    cls = jnp.broadcast_to(cls_token, (batch, 1, _H))
    return jnp.concatenate([cls, num_tokens, cat1, cat2], axis=1)


# speedup vs baseline: 1.7695x; 1.0469x over previous
import functools

import jax
import jax.numpy as jnp
from jax import lax
from jax.experimental import pallas as pl
from jax.experimental.pallas import tpu as pltpu
from jax.experimental.pallas import tpu_sc as plsc

_B = 4096
_NUM = 13
_NCAT = 26
_VOCAB = 100000
_H = 64

_NC = 2
_NS = 16
_NW = _NC * _NS
_NCH = _NCAT // 2
_PER_W = _B * _NCH // _NW
_CHUNK = 128
_NCHUNK = _PER_W // _CHUNK

_VB = 16384
_PPB = _VB // 2
_NVB = (_VOCAB + _VB - 1) // _VB
_PPF = _NVB * _PPB
_BB = 512


def _repack_body(xt_ref, out_ref):
    x = xt_ref[0]
    y = jnp.swapaxes(x, 0, 1)
    out_ref[0] = jnp.concatenate([y[: _PPB], y[_PPB:]], axis=1)


def _repack(tables_t, c0, nc):
    return pl.pallas_call(
        _repack_body,
        grid=(nc, _NVB),
        in_specs=[pl.BlockSpec((1, _H, _VB), lambda c, j: (c + c0, 0, j))],
        out_specs=pl.BlockSpec((1, _PPB, 2 * _H), lambda c, j: (c, j, 0)),
        out_shape=jax.ShapeDtypeStruct((nc, _PPF, 2 * _H), jnp.float32),
    )(tables_t)


def _mlp_body(x_ref, w1_ref, b1_ref, w2_ref, b2_ref, out_ref):
    x = x_ref[...]
    for n in range(_NUM):
        h = x[:, n:n + 1] * w1_ref[n:n + 1, :] + b1_ref[n:n + 1, :]
        h = 0.5 * h * (1.0 + lax.erf(h * 0.7071067811865476))
        t = jnp.dot(h, w2_ref[n], preferred_element_type=jnp.float32)
        out_ref[:, n, :] = t + b2_ref[n:n + 1, :]


def _num_tokens(x_num, W1, b1, W2, b2):
    return pl.pallas_call(
        _mlp_body,
        grid=(_B // _BB,),
        in_specs=[
            pl.BlockSpec((_BB, _NUM), lambda i: (i, 0)),
            pl.BlockSpec((_NUM, _H), lambda i: (0, 0)),
            pl.BlockSpec((_NUM, _H), lambda i: (0, 0)),
            pl.BlockSpec((_NUM, _H, _H), lambda i: (0, 0, 0)),
            pl.BlockSpec((_NUM, _H), lambda i: (0, 0)),
        ],
        out_specs=pl.BlockSpec((_BB, _NUM, _H), lambda i: (i, 0, 0)),
        out_shape=jax.ShapeDtypeStruct((_B, _NUM, _H), jnp.float32),
    )(x_num, W1, b1, W2, b2)


def _sc_gather_body(tpair_hbm, ge_hbm, out_hbm, ge_v, pidx_v, rows_v, out_v,
                    sems):
    wid = lax.axis_index("s") * _NC + lax.axis_index("c")
    base = pl.multiple_of(wid * _PER_W, _PER_W)
    pltpu.sync_copy(ge_hbm.at[pl.ds(base, _PER_W)], ge_v)

    def shift_body(k, carry):
        off = pl.multiple_of(k * 16, 16)
        pidx_v[pl.ds(off, 16)] = lax.shift_right_logical(
            ge_v[pl.ds(off, 16)], 1)
        return carry

    lax.fori_loop(0, _PER_W // 16, shift_body, 0)

    lanes = lax.iota(jnp.int32, 16)

    def fire(j, slot):
        off = pl.multiple_of(j * _CHUNK, _CHUNK)
        pltpu.async_copy(tpair_hbm.at[pidx_v.at[pl.ds(off, _CHUNK)]],
                         rows_v.at[slot], sems.at[slot])

    fire(0, 0)

    def chunk_body(j, carry):
        slot = lax.rem(j, 2)
        nxt = lax.rem(j + 1, 2)

        @pl.when(j + 1 < _NCHUNK)
        def _():
            fire(j + 1, nxt)

        pltpu.make_async_copy(tpair_hbm.at[pl.ds(0, _CHUNK)],
                              rows_v.at[slot], sems.at[slot]).wait()

        off = pl.multiple_of(j * _CHUNK, _CHUNK)
        rv = rows_v.at[slot]

        def grp_body(g, carry2):
            goff = pl.multiple_of(g * 16, 16)
            rowi = lanes + goff
            half = lax.bitwise_and(ge_v[pl.ds(off + goff, 16)], 1)
            colbase = half * _H
            for c in range(_H):
                vals = plsc.load_gather(rv, [rowi, colbase + c])
                plsc.store_scatter(out_v, [rowi, lanes * 0 + c], vals)
            return carry2

        lax.fori_loop(0, _CHUNK // 16, grp_body, 0)
        pltpu.sync_copy(out_v, out_hbm.at[pl.ds(base + off, _CHUNK)])
        return carry

    lax.fori_loop(0, _NCHUNK, chunk_body, 0)


def _sc_gather(tpair_flat, ge):
    mesh = plsc.VectorSubcoreMesh(core_axis_name="c", subcore_axis_name="s")
    run = functools.partial(
        pl.kernel,
        out_type=jax.ShapeDtypeStruct((_B * _NCH, _H), jnp.float32),
        mesh=mesh,
        scratch_types=[
            pltpu.VMEM((_PER_W,), jnp.int32),
            pltpu.VMEM((_PER_W,), jnp.int32),
            pltpu.VMEM((2, _CHUNK, 2 * _H), jnp.float32),
            pltpu.VMEM((_CHUNK, _H), jnp.float32),
            pltpu.SemaphoreType.DMA((2,)),
        ],
        compiler_params=pltpu.CompilerParams(use_tc_tiling_on_sc=True,
                                             needs_layout_passes=False),
    )(_sc_gather_body)
    return run(tpair_flat, ge)


def kernel(x_num, x_cat, W1, b1, W2, b2, tables, cls_token):
    batch = x_num.shape[0]
    tables_t = jnp.swapaxes(tables, 1, 2)

    v = x_cat
    pair = (v // _VB) * _PPB + (v % _PPB)
    half = (v // _PPB) % 2
    crange = jnp.arange(_NCH, dtype=jnp.int32)[None, :]
    ge1 = ((crange * _PPF + pair[:, : _NCH]) * 2
           + half[:, : _NCH]).reshape(-1)
    ge2 = ((crange * _PPF + pair[:, _NCH:]) * 2
           + half[:, _NCH:]).reshape(-1)

    tp1 = _repack(tables_t, 0, _NCH).reshape(_NCH * _PPF, 2 * _H)
    cat1 = _sc_gather(tp1, ge1).reshape(batch, _NCH, _H)
    tp2 = _repack(tables_t, _NCH, _NCH).reshape(_NCH * _PPF, 2 * _H)
    cat2 = _sc_gather(tp2, ge2).reshape(batch, _NCH, _H)

    num_tokens = _num_tokens(x_num, W1, b1, W2, b2)
    cls = jnp.broadcast_to(cls_token, (batch, 1, _H))
    return jnp.concatenate([cls, num_tokens, cat1, cat2], axis=1)


# speedup vs baseline: 2.0019x; 1.1313x over previous
import functools

import jax
import jax.numpy as jnp
from jax import lax
from jax.experimental import pallas as pl
from jax.experimental.pallas import tpu as pltpu
from jax.experimental.pallas import tpu_sc as plsc

_B = 4096
_NUM = 13
_NCAT = 26
_VOCAB = 100000
_H = 64

_NC = 2
_NS = 16
_NW = _NC * _NS
_NCH = _NCAT // 2
_PER_W = _B * _NCH // _NW
_CHUNK = 128
_NCHUNK = _PER_W // _CHUNK

_VB = 25088
_PPB = _VB // 2
_NVB = (_VOCAB + _VB - 1) // _VB
_PPF = _NVB * _PPB
_BB = 512


def _repack_body(xt_ref, out_ref):
    x = xt_ref[0]
    y = jnp.swapaxes(x, 0, 1)
    out_ref[0] = jnp.concatenate([y[: _PPB], y[_PPB:]], axis=1)


def _repack(tables_t, c0, nc):
    return pl.pallas_call(
        _repack_body,
        grid=(nc, _NVB),
        in_specs=[pl.BlockSpec((1, _H, _VB), lambda c, j: (c + c0, 0, j))],
        out_specs=pl.BlockSpec((1, _PPB, 2 * _H), lambda c, j: (c, j, 0)),
        out_shape=jax.ShapeDtypeStruct((nc, _PPF, 2 * _H), jnp.float32),
    )(tables_t)


def _mlp_body(x_ref, w1_ref, b1_ref, w2_ref, b2_ref, out_ref):
    x = x_ref[...]
    for n in range(_NUM):
        h = x[:, n:n + 1] * w1_ref[n:n + 1, :] + b1_ref[n:n + 1, :]
        h = 0.5 * h * (1.0 + lax.erf(h * 0.7071067811865476))
        t = jnp.dot(h, w2_ref[n], preferred_element_type=jnp.float32)
        out_ref[:, n, :] = t + b2_ref[n:n + 1, :]


def _num_tokens(x_num, W1, b1, W2, b2):
    return pl.pallas_call(
        _mlp_body,
        grid=(_B // _BB,),
        in_specs=[
            pl.BlockSpec((_BB, _NUM), lambda i: (i, 0)),
            pl.BlockSpec((_NUM, _H), lambda i: (0, 0)),
            pl.BlockSpec((_NUM, _H), lambda i: (0, 0)),
            pl.BlockSpec((_NUM, _H, _H), lambda i: (0, 0, 0)),
            pl.BlockSpec((_NUM, _H), lambda i: (0, 0)),
        ],
        out_specs=pl.BlockSpec((_BB, _NUM, _H), lambda i: (i, 0, 0)),
        out_shape=jax.ShapeDtypeStruct((_B, _NUM, _H), jnp.float32),
    )(x_num, W1, b1, W2, b2)


def _sc_gather_body(tpair_hbm, ge_hbm, out_hbm, ge_v, pidx_v, rows_v, out_v,
                    sems):
    wid = lax.axis_index("s") * _NC + lax.axis_index("c")
    base = pl.multiple_of(wid * _PER_W, _PER_W)
    pltpu.sync_copy(ge_hbm.at[pl.ds(base, _PER_W)], ge_v)

    def shift_body(k, carry):
        off = pl.multiple_of(k * 16, 16)
        pidx_v[pl.ds(off, 16)] = lax.shift_right_logical(
            ge_v[pl.ds(off, 16)], 1)
        return carry

    lax.fori_loop(0, _PER_W // 16, shift_body, 0)

    lanes = lax.iota(jnp.int32, 16)

    def fire(j, slot):
        off = pl.multiple_of(j * _CHUNK, _CHUNK)
        pltpu.async_copy(tpair_hbm.at[pidx_v.at[pl.ds(off, _CHUNK)]],
                         rows_v.at[slot], sems.at[slot])

    fire(0, 0)

    def chunk_body(j, carry):
        slot = lax.rem(j, 2)
        nxt = lax.rem(j + 1, 2)

        @pl.when(j + 1 < _NCHUNK)
        def _():
            fire(j + 1, nxt)

        pltpu.make_async_copy(tpair_hbm.at[pl.ds(0, _CHUNK)],
                              rows_v.at[slot], sems.at[slot]).wait()

        off = pl.multiple_of(j * _CHUNK, _CHUNK)
        rv = rows_v.at[slot]

        def grp_body(g, carry2):
            goff = pl.multiple_of(g * 16, 16)
            rowi = lanes + goff
            half = lax.bitwise_and(ge_v[pl.ds(off + goff, 16)], 1)
            colbase = half * _H
            for c in range(_H):
                vals = plsc.load_gather(rv, [rowi, colbase + c])
                plsc.store_scatter(out_v, [rowi, lanes * 0 + c], vals)
            return carry2

        lax.fori_loop(0, _CHUNK // 16, grp_body, 0)
        pltpu.sync_copy(out_v, out_hbm.at[pl.ds(base + off, _CHUNK)])
        return carry

    lax.fori_loop(0, _NCHUNK, chunk_body, 0)


def _sc_gather(tpair_flat, ge):
    mesh = plsc.VectorSubcoreMesh(core_axis_name="c", subcore_axis_name="s")
    run = functools.partial(
        pl.kernel,
        out_type=jax.ShapeDtypeStruct((_B * _NCH, _H), jnp.float32),
        mesh=mesh,
        scratch_types=[
            pltpu.VMEM((_PER_W,), jnp.int32),
            pltpu.VMEM((_PER_W,), jnp.int32),
            pltpu.VMEM((2, _CHUNK, 2 * _H), jnp.float32),
            pltpu.VMEM((_CHUNK, _H), jnp.float32),
            pltpu.SemaphoreType.DMA((2,)),
        ],
        compiler_params=pltpu.CompilerParams(use_tc_tiling_on_sc=True,
                                             needs_layout_passes=False),
    )(_sc_gather_body)
    return run(tpair_flat, ge)


def kernel(x_num, x_cat, W1, b1, W2, b2, tables, cls_token):
    batch = x_num.shape[0]
    tables_t = jnp.swapaxes(tables, 1, 2)

    v = x_cat
    pair = (v // _VB) * _PPB + (v % _PPB)
    half = (v // _PPB) % 2
    crange = jnp.arange(_NCH, dtype=jnp.int32)[None, :]
    ge1 = ((crange * _PPF + pair[:, : _NCH]) * 2
           + half[:, : _NCH]).reshape(-1)
    ge2 = ((crange * _PPF + pair[:, _NCH:]) * 2
           + half[:, _NCH:]).reshape(-1)

    tp1 = _repack(tables_t, 0, _NCH).reshape(_NCH * _PPF, 2 * _H)
    cat1 = _sc_gather(tp1, ge1).reshape(batch, _NCH, _H)
    tp2 = _repack(tables_t, _NCH, _NCH).reshape(_NCH * _PPF, 2 * _H)
    cat2 = _sc_gather(tp2, ge2).reshape(batch, _NCH, _H)

    num_tokens = _num_tokens(x_num, W1, b1, W2, b2)
    cls = jnp.broadcast_to(cls_token, (batch, 1, _H))
    return jnp.concatenate([cls, num_tokens, cat1, cat2], axis=1)
